# baseline (speedup 1.0000x reference)
import jax
import jax.numpy as jnp
from jax.experimental import pallas as pl
from jax.experimental.pallas import tpu as pltpu


def _seqpool_body(x_ref, w_ref, b_ref, o_ref):
    x = x_ref[...]
    logits = jnp.sum(x * w_ref[...], axis=2, keepdims=True) + b_ref[0]
    logits = logits - jnp.max(logits, axis=1, keepdims=True)
    e = jnp.exp(logits)
    p = e / jnp.sum(e, axis=1, keepdims=True)
    o_ref[...] = jnp.sum(p * x, axis=1, keepdims=True).astype(o_ref.dtype)


def _pick_batch_tile(B):
    for tb in range(min(128, B), 0, -8):
        if B % tb == 0:
            return tb
    return 1


def kernel(x, w, b):
    B, N, D = x.shape
    tb = _pick_batch_tile(B)
    grid = (B // tb,)
    out = pl.pallas_call(
        _seqpool_body,
        out_shape=jax.ShapeDtypeStruct((B, 1, D), x.dtype),
        grid=grid,
        in_specs=[
            pl.BlockSpec((tb, N, D), lambda i: (i, 0, 0)),
            pl.BlockSpec(memory_space=pltpu.MemorySpace.VMEM),
            pl.BlockSpec(memory_space=pltpu.MemorySpace.SMEM),
        ],
        out_specs=pl.BlockSpec((tb, 1, D), lambda i: (i, 0, 0)),
        compiler_params=pltpu.CompilerParams(
            dimension_semantics=("parallel",),
            vmem_limit_bytes=64 * 1024 * 1024,
        ),
    )(x, w, b)
    return out


# speedup vs baseline: 1.1595x; 1.1595x over previous
import jax
import jax.numpy as jnp
from jax.experimental import pallas as pl
from jax.experimental.pallas import tpu as pltpu


def _seqpool_body(x_ref, w_ref, b_ref, o_ref):
    x = x_ref[...]
    logits = jnp.sum(x * w_ref[...], axis=2, keepdims=True) + b_ref[0]
    logits = logits - jnp.max(logits, axis=1, keepdims=True)
    e = jnp.exp(logits)
    p = e / jnp.sum(e, axis=1, keepdims=True)
    o_ref[...] = jnp.sum(p * x, axis=1, keepdims=True).astype(o_ref.dtype)


def _pick_batch_tile(B):
    for tb in range(min(512, B), 0, -8):
        if B % tb == 0:
            return tb
    return 1


def kernel(x, w, b):
    B, N, D = x.shape
    tb = _pick_batch_tile(B)
    grid = (B // tb,)
    out = pl.pallas_call(
        _seqpool_body,
        out_shape=jax.ShapeDtypeStruct((B, 1, D), x.dtype),
        grid=grid,
        in_specs=[
            pl.BlockSpec((tb, N, D), lambda i: (i, 0, 0)),
            pl.BlockSpec(memory_space=pltpu.MemorySpace.VMEM),
            pl.BlockSpec(memory_space=pltpu.MemorySpace.SMEM),
        ],
        out_specs=pl.BlockSpec((tb, 1, D), lambda i: (i, 0, 0)),
        compiler_params=pltpu.CompilerParams(
            dimension_semantics=("parallel",),
            vmem_limit_bytes=64 * 1024 * 1024,
        ),
    )(x, w, b)
    return out
